# baseline (speedup 1.0000x reference)
import functools

import jax
import jax.numpy as jnp
from jax import lax
from jax.experimental import pallas as pl
from jax.experimental.pallas import tpu as pltpu
from jax.experimental.pallas import tpu_sc as plsc

_N_CLS = 1000
_D = 512
_P = 5
_C = 5
_T = 67
_M = _P + _C + _T

_info = plsc.get_sparse_core_info()
_NC = _info.num_cores
_NS = _info.num_subcores
_NW = _NC * _NS

_B = 200
_LANES = 16
_JROW = _D // _LANES

_mesh = plsc.VectorSubcoreMesh(core_axis_name="c", subcore_axis_name="s")


@functools.partial(
    pl.kernel,
    mesh=_mesh,
    out_type=jax.ShapeDtypeStruct((_M, _N_CLS, _D), jnp.float32),
    scratch_types=[
        pltpu.VMEM((_P, _D), jnp.float32),
        pltpu.VMEM((_B, _D), jnp.float32),
        pltpu.SemaphoreType.DMA,
        pltpu.SemaphoreType.DMA,
    ],
)
def _prompt_concat(prefix_hbm, ctx_hbm, token_hbm, out_hbm, pbuf, prep, in_sem, out_sem):
    core = lax.axis_index("c")
    sub = lax.axis_index("s")
    wid = sub * _NC + core

    @pl.when(wid < 3)
    def _():
        p0 = 3 * wid
        pltpu.async_copy(
            token_hbm.at[pl.ds(p0, 3)],
            out_hbm.at[pl.ds(_P + _C, _T)].at[pl.ds(p0, 3)],
            out_sem,
        )

    @pl.when(wid >= 3)
    def _():
        p0 = 9 + 2 * (wid - 3)
        pltpu.async_copy(
            token_hbm.at[pl.ds(p0, 2)],
            out_hbm.at[pl.ds(_P + _C, _T)].at[pl.ds(p0, 2)],
            out_sem,
        )

    @pl.when(wid == _NW - 1)
    def _():
        pltpu.async_copy(ctx_hbm, out_hbm.at[pl.ds(_P, _C)], out_sem)

    @pl.when(wid < _P)
    def _():
        pltpu.async_copy(prefix_hbm, pbuf, in_sem)
        pltpu.make_async_copy(prefix_hbm, pbuf, in_sem).wait()

        def fill(i, carry):
            for j in range(_JROW):
                sl = pl.ds(j * _LANES, _LANES)
                prep[i, sl] = pbuf[wid, sl]
            return carry

        lax.fori_loop(0, _B, fill, 0)
        for k in range(_N_CLS // _B):
            pltpu.async_copy(
                prep, out_hbm.at[wid, pl.ds(k * _B, _B)], out_sem
            )

    @pl.when(wid < 3)
    def _():
        pltpu.make_async_copy(
            token_hbm.at[pl.ds(0, 3)], out_hbm.at[pl.ds(_P + _C, 3)], out_sem
        ).wait()

    @pl.when(wid >= 3)
    def _():
        pltpu.make_async_copy(
            token_hbm.at[pl.ds(0, 2)], out_hbm.at[pl.ds(_P + _C, 2)], out_sem
        ).wait()

    @pl.when(wid == _NW - 1)
    def _():
        pltpu.make_async_copy(ctx_hbm, out_hbm.at[pl.ds(_P, _C)], out_sem).wait()

    @pl.when(wid < _P)
    def _():
        for k in range(_N_CLS // _B):
            pltpu.make_async_copy(
                prep, out_hbm.at[wid, pl.ds(0, _B)], out_sem
            ).wait()


def kernel(prefix, ctx, token):
    out_t = _prompt_concat(
        prefix,
        ctx.transpose(1, 0, 2),
        token.transpose(1, 0, 2),
    )
    return out_t.transpose(1, 0, 2)


# speedup vs baseline: 37.4492x; 37.4492x over previous
import functools

import jax
import jax.numpy as jnp
from jax import lax
from jax.experimental import pallas as pl
from jax.experimental.pallas import tpu as pltpu
from jax.experimental.pallas import tpu_sc as plsc

_N_CLS = 1000
_D = 512
_P = 5
_C = 5
_T = 67
_M = _P + _C + _T

_info = plsc.get_sparse_core_info()
_NC = _info.num_cores
_NS = _info.num_subcores
_NW = _NC * _NS

_ROWS = 32
_LAST = _N_CLS - _ROWS

_K = 3
_NCHUNK = _T // _K

_LANES = 16
_JROW = _D // _LANES

_mesh = plsc.VectorSubcoreMesh(core_axis_name="c", subcore_axis_name="s")


@functools.partial(
    pl.kernel,
    mesh=_mesh,
    out_type=jax.ShapeDtypeStruct((_M, _N_CLS, _D), jnp.float32),
    scratch_types=[
        pltpu.VMEM_SHARED((_NS, 2, _K, _ROWS, _D), jnp.float32),
        pltpu.VMEM((_P, _D), jnp.float32),
        pltpu.VMEM((_P, 8, _D), jnp.float32),
        pltpu.SemaphoreType.DMA,
        pltpu.SemaphoreType.DMA,
    ],
)
def _prompt_concat(
    prefix_hbm, ctx_hbm, token_hbm, out_hbm, sbuf, pbuf, prep, in_sem, out_sem
):
    core = lax.axis_index("c")
    sub = lax.axis_index("s")
    wid = sub * _NC + core
    start = pl.multiple_of(jnp.minimum(_ROWS * wid, _LAST), 8)
    rows = pl.ds(start, _ROWS)
    buf = sbuf.at[sub]

    def wait_in(k):
        pltpu.make_async_copy(
            token_hbm.at[pl.ds(0, k), pl.ds(0, _ROWS)],
            buf.at[0, pl.ds(0, k)],
            in_sem,
        ).wait()

    def wait_out(k):
        pltpu.make_async_copy(
            buf.at[0, pl.ds(0, k)],
            out_hbm.at[pl.ds(0, k), pl.ds(0, _ROWS)],
            out_sem,
        ).wait()

    pltpu.async_copy(prefix_hbm, pbuf, in_sem)

    pltpu.async_copy(ctx_hbm.at[pl.ds(0, 3), rows], buf.at[0], in_sem)
    pltpu.async_copy(ctx_hbm.at[pl.ds(3, 2), rows], buf.at[1, pl.ds(0, 2)], in_sem)
    pltpu.make_async_copy(prefix_hbm, pbuf, in_sem).wait()
    wait_in(3)
    pltpu.async_copy(buf.at[0], out_hbm.at[pl.ds(_P, 3), rows], out_sem)
    wait_in(2)
    pltpu.async_copy(
        buf.at[1, pl.ds(0, 2)], out_hbm.at[pl.ds(_P + 3, 2), rows], out_sem
    )
    wait_out(3)
    wait_out(2)

    pltpu.async_copy(token_hbm.at[pl.ds(0, _K), rows], buf.at[0], in_sem)

    def body(i, carry):
        slot = lax.rem(i, 2)
        wait_in(_K)
        pltpu.async_copy(
            buf.at[slot],
            out_hbm.at[pl.ds(_P + _C + _K * i, _K), rows],
            out_sem,
        )

        @pl.when(i + 1 < _NCHUNK)
        def _():
            @pl.when(i >= 1)
            def _():
                wait_out(_K)

            pltpu.async_copy(
                token_hbm.at[pl.ds(_K * (i + 1), _K), rows],
                buf.at[1 - slot],
                in_sem,
            )

        return carry

    lax.fori_loop(0, _NCHUNK, body, 0)

    wait_out(_K)

    pltpu.async_copy(
        token_hbm.at[pl.ds(_K * _NCHUNK, 1), rows], buf.at[0, pl.ds(0, 1)], in_sem
    )
    wait_in(1)
    pltpu.async_copy(
        buf.at[0, pl.ds(0, 1)], out_hbm.at[pl.ds(_M - 1, 1), rows], out_sem
    )

    for r in range(_P):

        def fill(j, carry):
            sl = pl.ds(j * _LANES, _LANES)
            v = pbuf[r, sl]
            for r2 in range(8):
                prep[r, r2, sl] = v
            return carry

        lax.fori_loop(0, _JROW, fill, 0)
        for k in range(4):
            pltpu.async_copy(
                prep.at[r],
                out_hbm.at[r, pl.ds(start + 8 * k, 8)],
                out_sem,
            )

    wait_out(_K)
    wait_out(1)
    for r in range(_P):
        for k in range(4):
            pltpu.make_async_copy(
                prep.at[r], out_hbm.at[r, pl.ds(start, 8)], out_sem
            ).wait()


def kernel(prefix, ctx, token):
    out_t = _prompt_concat(
        prefix,
        ctx.transpose(1, 0, 2),
        token.transpose(1, 0, 2),
    )
    return out_t.transpose(1, 0, 2)
